# baseline (speedup 1.0000x reference)
import functools

import jax
import jax.numpy as jnp
from jax import lax
from jax.experimental import pallas as pl
from jax.experimental.pallas import tpu as pltpu
from jax.experimental.pallas import tpu_sc as plsc

_NC = 2
_CHUNK = 256
_NB = 4
_W = 2


def kernel(x, pe):
    seq_len = x.shape[1]
    d = pe.shape[1]
    rows_w = seq_len // _NC
    nchunks = rows_w // _CHUNK

    mesh = plsc.ScalarSubcoreMesh(axis_name="c", num_cores=_NC)

    @functools.partial(
        pl.kernel,
        mesh=mesh,
        out_type=jax.ShapeDtypeStruct((seq_len, d), jnp.float32),
        scratch_types=[pltpu.VMEM_SHARED((_NB, _CHUNK, d), jnp.float32)]
        + [pltpu.SemaphoreType.DMA] * (2 * _NB),
    )
    def copy_k(pe_hbm, out_hbm, buf, *sems):
        s_in, s_out = sems[:_NB], sems[_NB:]
        base = lax.axis_index("c") * rows_w

        def in_copy(i):
            b = i % _NB
            return pltpu.make_async_copy(
                pe_hbm.at[pl.ds(base + i * _CHUNK, _CHUNK)], buf.at[b], s_in[b])

        def out_copy(i):
            b = i % _NB
            return pltpu.make_async_copy(
                buf.at[b], out_hbm.at[pl.ds(base + i * _CHUNK, _CHUNK)], s_out[b])

        for j in range(min(_NB, nchunks)):
            in_copy(j).start()
        for i in range(nchunks):
            in_copy(i).wait()
            out_copy(i).start()
            j = i - _W
            if j >= 0:
                out_copy(j).wait()
                if j + _NB < nchunks:
                    in_copy(j + _NB).start()
        for i in range(max(nchunks - _W, 0), nchunks):
            out_copy(i).wait()

    return copy_k(pe)[None]
